# baseline (speedup 1.0000x reference)
import functools

import jax
import jax.numpy as jnp
from jax import lax
from jax.experimental import pallas as pl
from jax.experimental.pallas import tpu as pltpu
from jax.experimental.pallas import tpu_sc as plsc

NC = 2
NS = 16
NW = NC * NS

EMB_D = 64
PAD_D = 128
BT = 128
L = 16


def _gather_kernel(batch: int, seq: int, n_rows: int):
    mesh = plsc.VectorSubcoreMesh(core_axis_name="c", subcore_axis_name="s",
                                  num_cores=NC, num_subcores=NS)
    n_btiles = batch // BT
    assert n_btiles == NW

    @functools.partial(
        pl.kernel,
        out_type=jax.ShapeDtypeStruct((seq, EMB_D // 8, n_btiles, 8, BT),
                                      jnp.float32),
        mesh=mesh,
        scratch_types=[
            pltpu.VMEM((seq, BT), jnp.int32),
            pltpu.VMEM((2, BT, PAD_D), jnp.float32),
            pltpu.VMEM((2, EMB_D // 8, 8, BT), jnp.float32),
            pltpu.SemaphoreType.DMA,
            pltpu.SemaphoreType.DMA,
        ],
        compiler_params=pltpu.CompilerParams(needs_layout_passes=False),
    )
    def body(idst_hbm, table_hbm, out_hbm, idx_v, rows_v, trows_v, gsem, osem):
        wid = lax.axis_index("s") * NC + lax.axis_index("c")
        pltpu.sync_copy(idst_hbm.at[:, pl.ds(wid * BT, BT)], idx_v)

        def fire_gather(s, buf):
            pltpu.async_copy(table_hbm.at[idx_v.at[s]], rows_v.at[buf], gsem)

        def drain_gather(s, buf):
            pltpu.make_async_copy(
                table_hbm.at[idx_v.at[s]], rows_v.at[buf], gsem).wait()

        fire_gather(0, 0)
        drain_gather(0, 0)

        base16 = lax.iota(jnp.int32, L)

        @pl.loop(0, seq, step=2)
        def _(s0):
            for k in range(2):
                s = s0 + k
                nbuf = 1 - k

                @pl.when(s < seq - 1)
                def _():
                    fire_gather(s + 1, nbuf)

                rbuf = rows_v.at[k]
                tbuf = trows_v.at[k]
                for d in range(EMB_D):
                    col = jnp.full((L,), d, jnp.int32)
                    for j in range(BT // L):
                        vals = plsc.load_gather(
                            rbuf, [base16 + (j * L), col])
                        tbuf[d // 8, d % 8, pl.ds(j * L, L)] = vals

                out_cp = pltpu.async_copy(tbuf, out_hbm.at[s, :, wid], osem)

                @pl.when(s < seq - 1)
                def _():
                    drain_gather(s + 1, nbuf)
                out_cp.wait()

    return body


def kernel(codec_ids, table):
    batch, seq = codec_ids.shape
    n_rows, emb_d = table.shape
    assert emb_d == EMB_D and batch == NW * BT
    ids_t = codec_ids.T.astype(jnp.int32)
    table_pad = jnp.pad(table, ((0, 0), (0, PAD_D - EMB_D)))
    out5 = _gather_kernel(batch, seq, n_rows)(ids_t, table_pad)
    return (out5.transpose(2, 4, 0, 1, 3)
            .reshape(batch, seq, EMB_D))


# speedup vs baseline: 2.1548x; 2.1548x over previous
import functools

import jax
import jax.numpy as jnp
from jax import lax
from jax.experimental import pallas as pl
from jax.experimental.pallas import tpu as pltpu
from jax.experimental.pallas import tpu_sc as plsc

NC = 2
NS = 16
NW = NC * NS

EMB_D = 64
PAD_D = 128
GRP = 128
G_PER_IT = 8
ROWS_PER_IT = GRP * G_PER_IT


def _gather_kernel(n: int, n_rows: int):
    mesh = plsc.VectorSubcoreMesh(core_axis_name="c", subcore_axis_name="s",
                                  num_cores=NC, num_subcores=NS)
    n_per_w = n // NW
    n_iters = n_per_w // ROWS_PER_IT

    @functools.partial(
        pl.kernel,
        out_type=jax.ShapeDtypeStruct((n, PAD_D), jnp.float32),
        mesh=mesh,
        scratch_types=[
            pltpu.VMEM((n_per_w,), jnp.int32),
            pltpu.VMEM((ROWS_PER_IT, EMB_D), jnp.float32),
            pltpu.SemaphoreType.DMA,
        ],
        compiler_params=pltpu.CompilerParams(use_tc_tiling_on_sc=False),
    )
    def body(ids_hbm, table_hbm, out_hbm, idx_v, rows_v, gsem):
        wid = lax.axis_index("s") * NC + lax.axis_index("c")
        row_base = wid * n_per_w
        pltpu.sync_copy(ids_hbm.at[pl.ds(row_base, n_per_w)], idx_v)

        @pl.loop(0, n_iters)
        def _(it):
            copies = []
            for g in range(G_PER_IT):
                copies.append(pltpu.async_copy(
                    table_hbm.at[idx_v.at[pl.ds(it * ROWS_PER_IT + g * GRP,
                                                GRP)]],
                    rows_v.at[pl.ds(g * GRP, GRP)],
                    gsem,
                ))
            for c in copies:
                c.wait()
            pltpu.sync_copy(
                rows_v,
                out_hbm.at[pl.ds(row_base + it * ROWS_PER_IT, ROWS_PER_IT),
                           pl.ds(0, EMB_D)],
            )

    return body


def kernel(codec_ids, table):
    batch, seq = codec_ids.shape
    n_rows, emb_d = table.shape
    n = batch * seq
    assert emb_d == EMB_D and n % (NW * ROWS_PER_IT) == 0
    ids = codec_ids.astype(jnp.int32).reshape(-1)
    out_pad = _gather_kernel(n, n_rows)(ids, table)
    return out_pad.reshape(batch, seq, PAD_D)[:, :, :EMB_D]
